# baseline (speedup 1.0000x reference)
import functools

import jax
import jax.numpy as jnp
from jax import lax
from jax.experimental import pallas as pl
from jax.experimental.pallas import tpu as pltpu
from jax.experimental.pallas import tpu_sc as plsc

B = 4096
L = 50
DIM = 128
NC = 2
NS = 16
NW = NC * NS
P = 2
BP = B // P
SENT_W = BP // NW
CS = 2
NCHUNK = SENT_W // CS
NBUF = 4
NGRP = NCHUNK // NBUF


def _emb_body(ids_hbm, table_hbm, out_hbm, idx_v, rows_v, *sems):
    gsems = sems[:NBUF]
    ssems = sems[NBUF:]
    wid = lax.axis_index("s") * NC + lax.axis_index("c")
    base = wid * SENT_W
    pltpu.sync_copy(ids_hbm.at[wid], idx_v)

    def gstart(j, b):
        for s in range(CS):
            pltpu.make_async_copy(
                table_hbm.at[idx_v.at[j * CS + s]], rows_v.at[b, s],
                gsems[b]).start()

    def gwait(b):
        for s in range(CS):
            pltpu.make_async_copy(
                table_hbm.at[idx_v.at[0]], rows_v.at[b, s], gsems[b]).wait()

    def sstart(j, b):
        pltpu.make_async_copy(
            rows_v.at[b], out_hbm.at[pl.ds(base + j * CS, CS)],
            ssems[b]).start()

    def swait(b):
        pltpu.make_async_copy(
            rows_v.at[b], out_hbm.at[pl.ds(base, CS)], ssems[b]).wait()

    for b in range(NBUF):
        gstart(b, b)

    def body(g, carry):
        j0 = g * NBUF
        for b in range(NBUF):
            gwait(b)
            sstart(j0 + b, b)
        for b in range(NBUF):
            swait(b)
            gstart(j0 + NBUF + b, b)
        return carry

    lax.fori_loop(0, NGRP - 1, body, 0)

    j0 = (NGRP - 1) * NBUF
    for b in range(NBUF):
        gwait(b)
        sstart(j0 + b, b)
    for b in range(NBUF):
        swait(b)


def kernel(word_ids, table):
    mesh = plsc.VectorSubcoreMesh(core_axis_name="c", subcore_axis_name="s")
    emb = functools.partial(
        pl.kernel,
        mesh=mesh,
        out_type=jax.ShapeDtypeStruct((BP, L, DIM), jnp.float32),
        scratch_types=[
            pltpu.VMEM((SENT_W, L), jnp.int32),
            pltpu.VMEM((NBUF, CS, L, DIM), jnp.float32),
        ] + [pltpu.SemaphoreType.DMA] * (2 * NBUF),
    )(_emb_body)
    out = jnp.zeros((B, L, DIM), jnp.float32)
    for p in range(P):
        ids_p = word_ids[p * BP:(p + 1) * BP].reshape(NW, SENT_W, L)
        out = lax.dynamic_update_slice(out, emb(ids_p, table), (p * BP, 0, 0))
    return out


# speedup vs baseline: 2.9466x; 2.9466x over previous
import functools

import jax
import jax.numpy as jnp
from jax import lax
from jax.experimental import pallas as pl
from jax.experimental.pallas import tpu as pltpu
from jax.experimental.pallas import tpu_sc as plsc

B = 4096
L = 50
DIM = 128
NC = 2
NS = 16
NW = NC * NS
SENT_W = B // NW
NBUF = 5
NGRP = L // NBUF


def _emb_body(ids_hbm, table_hbm, out_hbm, idx_v, rows_v, *sems):
    gsems = sems[:NBUF]
    ssems = sems[NBUF:]
    wid = lax.axis_index("s") * NC + lax.axis_index("c")
    base = wid * SENT_W
    pltpu.sync_copy(ids_hbm.at[wid], idx_v)

    def gstart(l, b):
        pltpu.make_async_copy(
            table_hbm.at[idx_v.at[l]], rows_v.at[b], gsems[b]).start()

    def gwait(b):
        pltpu.make_async_copy(
            table_hbm.at[idx_v.at[0]], rows_v.at[b], gsems[b]).wait()

    def sstart(l, b):
        pltpu.make_async_copy(
            rows_v.at[b], out_hbm.at[l, pl.ds(base, SENT_W)],
            ssems[b]).start()

    def swait(b):
        pltpu.make_async_copy(
            rows_v.at[b], out_hbm.at[0, pl.ds(base, SENT_W)],
            ssems[b]).wait()

    for b in range(NBUF):
        gstart(b, b)

    def body(g, carry):
        l0 = g * NBUF
        for b in range(NBUF):
            gwait(b)
            sstart(l0 + b, b)
        for b in range(NBUF):
            swait(b)
            gstart(l0 + NBUF + b, b)
        return carry

    lax.fori_loop(0, NGRP - 1, body, 0)

    l0 = (NGRP - 1) * NBUF
    for b in range(NBUF):
        gwait(b)
        sstart(l0 + b, b)
    for b in range(NBUF):
        swait(b)


def kernel(word_ids, table):
    ids_r = word_ids.reshape(NW, SENT_W, L).transpose(0, 2, 1)
    mesh = plsc.VectorSubcoreMesh(core_axis_name="c", subcore_axis_name="s")
    emb = functools.partial(
        pl.kernel,
        mesh=mesh,
        out_type=jax.ShapeDtypeStruct((L, B, DIM), jnp.float32),
        scratch_types=[
            pltpu.VMEM((L, SENT_W), jnp.int32),
            pltpu.VMEM((NBUF, SENT_W, DIM), jnp.float32),
        ] + [pltpu.SemaphoreType.DMA] * (2 * NBUF),
    )(_emb_body)
    out_t = emb(ids_r, table)
    return out_t.transpose(1, 0, 2)


# speedup vs baseline: 3.0101x; 1.0215x over previous
import functools

import jax
import jax.numpy as jnp
from jax import lax
from jax.experimental import pallas as pl
from jax.experimental.pallas import tpu as pltpu
from jax.experimental.pallas import tpu_sc as plsc

B = 4096
L = 50
DIM = 128
NC = 2
NS = 16
NW = NC * NS
SENT_W = B // NW
HALF = SENT_W // 2
NBUF = 10
NGRP = 2 * L // NBUF


def _emb_body(ids_hbm, table_hbm, out_hbm, idx_v, rows_v, *sems):
    gsems = sems[:NBUF]
    ssems = sems[NBUF:]
    wid = lax.axis_index("s") * NC + lax.axis_index("c")
    base = wid * SENT_W
    pltpu.sync_copy(ids_hbm.at[wid], idx_v)

    def gstart(l, h, b):
        pltpu.make_async_copy(
            table_hbm.at[idx_v.at[l, pl.ds(h * HALF, HALF)]], rows_v.at[b],
            gsems[b]).start()

    def gwait(b):
        pltpu.make_async_copy(
            table_hbm.at[idx_v.at[0, pl.ds(0, HALF)]], rows_v.at[b],
            gsems[b]).wait()

    def sstart(l, h, b):
        pltpu.make_async_copy(
            rows_v.at[b], out_hbm.at[l, pl.ds(base + h * HALF, HALF)],
            ssems[b]).start()

    def swait(b):
        pltpu.make_async_copy(
            rows_v.at[b], out_hbm.at[0, pl.ds(base, HALF)],
            ssems[b]).wait()

    LG = NBUF // 2
    for b in range(NBUF):
        gstart(b // 2, b % 2, b)

    def body(g, carry):
        l0 = g * LG
        for b in range(NBUF):
            gwait(b)
            sstart(l0 + b // 2, b % 2, b)
        for b in range(NBUF):
            swait(b)
            gstart(l0 + LG + b // 2, b % 2, b)
        return carry

    lax.fori_loop(0, NGRP - 1, body, 0)

    l0 = (NGRP - 1) * LG
    for b in range(NBUF):
        gwait(b)
        sstart(l0 + b // 2, b % 2, b)
    for b in range(NBUF):
        swait(b)


def kernel(word_ids, table):
    ids_r = word_ids.reshape(NW, SENT_W, L).transpose(0, 2, 1)
    mesh = plsc.VectorSubcoreMesh(core_axis_name="c", subcore_axis_name="s")
    emb = functools.partial(
        pl.kernel,
        mesh=mesh,
        out_type=jax.ShapeDtypeStruct((L, B, DIM), jnp.float32),
        scratch_types=[
            pltpu.VMEM((L, SENT_W), jnp.int32),
            pltpu.VMEM((NBUF, HALF, DIM), jnp.float32),
        ] + [pltpu.SemaphoreType.DMA] * (2 * NBUF),
    )(_emb_body)
    out_t = emb(ids_r, table)
    return out_t.transpose(1, 0, 2)
